# baseline (speedup 1.0000x reference)
import jax
import jax.numpy as jnp
from jax.experimental import pallas as pl

D_MODEL = 2048
K_CODES = 512
CODE_DIM = 64
TAU = 1.0

_BG = 4096
_BM = 512
_BN = 2048


def _quantize_block(z_ref, ct_ref, ca_ref, q_ref):
    z = z_ref[...]
    ct = ct_ref[...]
    c2 = 0.25 * jnp.sum(ct * ct, axis=0, keepdims=True)
    l = (jnp.dot(z.astype(jnp.bfloat16), ct.astype(jnp.bfloat16),
                 preferred_element_type=jnp.float32) - c2) * (1.0 / TAU)
    e = jnp.exp(l.astype(jnp.bfloat16))
    qs = jnp.dot(e, ca_ref[...], preferred_element_type=jnp.float32)
    q_ref[...] = qs[:, :CODE_DIM] / qs[:, CODE_DIM:CODE_DIM + 1]


def _matmul_block(x_ref, w_ref, o_ref):
    o_ref[...] = jnp.dot(x_ref[...], w_ref[...],
                         preferred_element_type=jnp.float32)


def kernel(x, subspace_params, centroids):
    z = subspace_params.reshape(-1, CODE_DIM)
    g = z.shape[0]

    ct = 2.0 * centroids.T
    ca = jnp.zeros((K_CODES, 2 * CODE_DIM), jnp.float32)
    ca = ca.at[:, :CODE_DIM].set(centroids).at[:, CODE_DIM].set(1.0)
    ca = ca.astype(jnp.bfloat16)

    q = pl.pallas_call(
        _quantize_block,
        grid=(g // _BG,),
        in_specs=[
            pl.BlockSpec((_BG, CODE_DIM), lambda i: (i, 0)),
            pl.BlockSpec((CODE_DIM, K_CODES), lambda i: (0, 0)),
            pl.BlockSpec((K_CODES, 2 * CODE_DIM), lambda i: (0, 0)),
        ],
        out_specs=pl.BlockSpec((_BG, CODE_DIM), lambda i: (i, 0)),
        out_shape=jax.ShapeDtypeStruct((g, CODE_DIM), jnp.float32),
    )(z, ct, ca)

    w = q.reshape(D_MODEL, D_MODEL)

    m = x.shape[0]
    out = pl.pallas_call(
        _matmul_block,
        grid=(m // _BM, D_MODEL // _BN),
        in_specs=[
            pl.BlockSpec((_BM, D_MODEL), lambda i, j: (i, 0)),
            pl.BlockSpec((D_MODEL, _BN), lambda i, j: (0, j)),
        ],
        out_specs=pl.BlockSpec((_BM, _BN), lambda i, j: (i, j)),
        out_shape=jax.ShapeDtypeStruct((m, D_MODEL), jnp.float32),
    )(x, w)
    return out


# speedup vs baseline: 1.0039x; 1.0039x over previous
import jax
import jax.numpy as jnp
from jax.experimental import pallas as pl

D_MODEL = 2048
K_CODES = 512
CODE_DIM = 64
TAU = 1.0

_BG = 8192
_BM = 512
_BN = 2048


def _quantize_block(z_ref, ct_ref, ca_ref, q_ref):
    z = z_ref[...]
    ct = ct_ref[...]
    c2 = 0.25 * jnp.sum(ct * ct, axis=0, keepdims=True)
    l = (jnp.dot(z.astype(jnp.bfloat16), ct.astype(jnp.bfloat16),
                 preferred_element_type=jnp.float32) - c2) * (1.0 / TAU)
    e = jnp.exp(l.astype(jnp.bfloat16))
    qs = jnp.dot(e, ca_ref[...], preferred_element_type=jnp.float32)
    q_ref[...] = qs[:, :CODE_DIM] / qs[:, CODE_DIM:CODE_DIM + 1]


def _matmul_block(x_ref, w_ref, o_ref):
    o_ref[...] = jnp.dot(x_ref[...], w_ref[...],
                         preferred_element_type=jnp.float32)


def kernel(x, subspace_params, centroids):
    z = subspace_params.reshape(-1, CODE_DIM)
    g = z.shape[0]

    ct = 2.0 * centroids.T
    ca = jnp.zeros((K_CODES, 2 * CODE_DIM), jnp.float32)
    ca = ca.at[:, :CODE_DIM].set(centroids).at[:, CODE_DIM].set(1.0)
    ca = ca.astype(jnp.bfloat16)

    q = pl.pallas_call(
        _quantize_block,
        grid=(g // _BG,),
        in_specs=[
            pl.BlockSpec((_BG, CODE_DIM), lambda i: (i, 0)),
            pl.BlockSpec((CODE_DIM, K_CODES), lambda i: (0, 0)),
            pl.BlockSpec((K_CODES, 2 * CODE_DIM), lambda i: (0, 0)),
        ],
        out_specs=pl.BlockSpec((_BG, CODE_DIM), lambda i: (i, 0)),
        out_shape=jax.ShapeDtypeStruct((g, CODE_DIM), jnp.float32),
    )(z, ct, ca)

    w = q.reshape(D_MODEL, D_MODEL)

    m = x.shape[0]
    out = pl.pallas_call(
        _matmul_block,
        grid=(m // _BM, D_MODEL // _BN),
        in_specs=[
            pl.BlockSpec((_BM, D_MODEL), lambda i, j: (i, 0)),
            pl.BlockSpec((D_MODEL, _BN), lambda i, j: (0, j)),
        ],
        out_specs=pl.BlockSpec((_BM, _BN), lambda i, j: (i, j)),
        out_shape=jax.ShapeDtypeStruct((m, D_MODEL), jnp.float32),
    )(x, w)
    return out
